# baseline (speedup 1.0000x reference)
import functools

import jax
import jax.numpy as jnp
from jax import lax
from jax.experimental import pallas as pl
from jax.experimental.pallas import tpu as pltpu
from jax.experimental.pallas import tpu_sc as plsc

_MXL = 100.0
_MNL = -100.0
_NP = 8
_NN = 64
_BIG = 3.4e38
_L16 = 16


def _vec16(x):
    return jnp.full((_L16,), x, jnp.float32)


_IOTA = lambda: lax.broadcasted_iota(jnp.int32, (_L16,), 0)


def _extract(v, j, pad):
    return jnp.min(jnp.where(_IOTA() == j, v, pad))


def _sc_body(x_hbm, t_hbm, thr_hbm, out_hbm, xbuf, tbuf, thrbuf,
             xbuf2, tbuf2, thrbuf2, nbuf, pbuf, obuf,
             semx, semt, semh, semx2, semt2, semh2,
             *, chunk_base, chunk_rows):
    l = x_hbm.shape[1]
    nvec = l // _L16
    wid = lax.axis_index("s") * 2 + lax.axis_index("c")
    rpw = chunk_rows // 32
    iota = _IOTA()

    def row_compute(acc, xbuf, tbuf, thrbuf):
        thv = thrbuf[pl.ds(0, _L16)]
        tau_n = jnp.min(jnp.where(iota == 0, thv, _BIG))
        tau_p = jnp.min(jnp.where(iota == 1, thv, _BIG))

        nvec2 = nvec // 2
        stride = (l >> 1) + _L16

        @plsc.parallel_loop(0, nvec2, unroll=8,
                            carry=(jnp.int32(0),) * 4)
        def p2(i, carry):
            offs = list(carry)
            for h in range(2):
                xv = xbuf[pl.ds((h * nvec2 + i) * _L16, _L16)]
                tv = tbuf[pl.ds((h * nvec2 + i) * _L16, _L16)]
                m1 = tv == 1.0
                sm = jnp.where(m1, _MNL, xv)
                sp = jnp.where(m1, xv, _MXL)
                mn = sm >= tau_n
                mp = sp <= tau_p
                plsc.store_compressed(
                    nbuf.at[pl.ds(h * stride + offs[2 * h], _L16)],
                    sm, mask=mn)
                plsc.store_compressed(
                    pbuf.at[pl.ds(h * stride + offs[2 * h + 1], _L16)],
                    sp, mask=mp)
                offs[2 * h] += plsc.all_reduce_population_count(mn)[0]
                offs[2 * h + 1] += plsc.all_reduce_population_count(mp)[0]
            return tuple(offs)

        cn0, cp0, cn1, cp1 = p2
        nbuf[pl.ds(cn0, _L16)] = _vec16(-_BIG)
        pbuf[pl.ds(cp0, _L16)] = _vec16(_BIG)
        nbuf[pl.ds(stride + cn1, _L16)] = _vec16(-_BIG)
        pbuf[pl.ds(stride + cp1, _L16)] = _vec16(_BIG)

        def casc_n(base):
            def step(i, ts):
                s = jnp.sort(-nbuf[pl.ds(base + i * _L16, _L16)])
                for k in range(4):
                    r = lax.rev(s, (0,))
                    lo = jnp.minimum(ts[k], r)
                    hi = jnp.maximum(ts[k], r)
                    ts = ts[:k] + (jnp.sort(lo),) + ts[k + 1:]
                    s = jnp.sort(hi)
                return ts
            return step

        tn = plsc.parallel_loop(0, (cn0 + _L16 - 1) >> 4, unroll=2,
                                carry=(_vec16(_BIG),) * 4)(casc_n(0))
        tn = plsc.parallel_loop(0, (cn1 + _L16 - 1) >> 4, unroll=2,
                                carry=tn)(casc_n(stride))

        def casc_p(base):
            def step(i, t0):
                s = jnp.sort(pbuf[pl.ds(base + i * _L16, _L16)])
                return jnp.sort(jnp.minimum(t0, lax.rev(s, (0,))))
            return step

        tp = plsc.parallel_loop(0, (cp0 + _L16 - 1) >> 4, unroll=2,
                                carry=_vec16(_BIG))(casc_p(0))
        tp = plsc.parallel_loop(0, (cp1 + _L16 - 1) >> 4, unroll=2,
                                carry=tp)(casc_p(stride))

        maxneg = -jnp.min(tn[0])
        for j in range(_NP):
            sp = _extract(tp, j, _BIG)
            mp_ = jnp.maximum(maxneg + 1.0 - sp, 0.0)
            num_v = _vec16(0.0)
            den_v = _vec16(0.0)
            for k in range(4):
                lossv = jnp.maximum(-tn[k] + (1.0 - sp), 0.0)
                prob = jnp.where(lossv > 0.0, lossv, _MNL)
                e = jnp.exp(prob - mp_)
                num_v = num_v + e * lossv
                den_v = den_v + e
            num = jnp.sum(num_v)
            den = jnp.sum(den_v)
            q = jnp.where(iota == 0, num, 0.0) / jnp.where(iota == 0, den, 1.0)
            acc = acc + jnp.where((iota == 0) & (num > 0.0), q, 0.0)
        return acc

    def start(row, xb, tb, hb, sx, st, sh):
        pltpu.make_async_copy(x_hbm.at[chunk_base + row], xb, sx).start()
        pltpu.make_async_copy(t_hbm.at[chunk_base + row], tb, st).start()
        pltpu.make_async_copy(thr_hbm.at[row], hb, sh).start()

    def wait(row, xb, tb, hb, sx, st, sh):
        pltpu.make_async_copy(x_hbm.at[chunk_base + row], xb, sx).wait()
        pltpu.make_async_copy(t_hbm.at[chunk_base + row], tb, st).wait()
        pltpu.make_async_copy(thr_hbm.at[row], hb, sh).wait()

    base = wid * rpw
    start(base, xbuf, tbuf, thrbuf, semx, semt, semh)

    def pair_step(k, acc):
        ra = base + 2 * k
        rb = ra + 1
        start(rb, xbuf2, tbuf2, thrbuf2, semx2, semt2, semh2)
        wait(ra, xbuf, tbuf, thrbuf, semx, semt, semh)
        acc = row_compute(acc, xbuf, tbuf, thrbuf)
        start(jnp.minimum(ra + 2, base + rpw - 1), xbuf, tbuf, thrbuf,
              semx, semt, semh)
        wait(rb, xbuf2, tbuf2, thrbuf2, semx2, semt2, semh2)
        return row_compute(acc, xbuf2, tbuf2, thrbuf2)

    acc = lax.fori_loop(0, rpw // 2, pair_step, _vec16(0.0))
    wait(base, xbuf, tbuf, thrbuf, semx, semt, semh)
    obuf[...] = acc
    pltpu.sync_copy(obuf, out_hbm.at[pl.ds(wid * _L16, _L16)])


def _thr_body(out_ref, tgt_ref, thr_ref):
    x = out_ref[...]
    r, l = x.shape
    tmask = tgt_ref[...] == 1.0
    sim_m = jnp.where(tmask, _MNL, x)
    sim_p = jnp.where(tmask, x, _MXL)
    tau_n = jnp.min(jnp.max(sim_m.reshape(r, 64, l // 64), axis=2), axis=1)
    tau_p = jnp.max(jnp.min(sim_p.reshape(r, 8, l // 8), axis=2), axis=1)
    lane = lax.broadcasted_iota(jnp.int32, (r, 128), 1)
    thr_ref[...] = jnp.where(lane == 0, tau_n[:, None],
                             jnp.where(lane == 1, tau_p[:, None], 0.0))


_NCHUNK = 4


@jax.jit
def kernel(output, target):
    b, l = output.shape
    r = 16
    cb = b // _NCHUNK
    mesh = plsc.VectorSubcoreMesh(core_axis_name="c", subcore_axis_name="s")

    partials = []
    for c in range(_NCHUNK):
        thr_c = pl.pallas_call(
            _thr_body,
            grid=(cb // r,),
            in_specs=[
                pl.BlockSpec((r, l), lambda i, _c=c: (i + _c * (cb // r), 0)),
                pl.BlockSpec((r, l), lambda i, _c=c: (i + _c * (cb // r), 0)),
            ],
            out_specs=pl.BlockSpec((r, 128), lambda i: (i, 0)),
            out_shape=jax.ShapeDtypeStruct((cb, 128), jnp.float32),
        )(output, target)

        fn = functools.partial(
            pl.kernel,
            mesh=mesh,
            out_type=jax.ShapeDtypeStruct((32 * _L16,), jnp.float32),
            compiler_params=pltpu.CompilerParams(needs_layout_passes=False),
            scratch_types=[
                pltpu.VMEM((l,), jnp.float32),
                pltpu.VMEM((l,), jnp.float32),
                pltpu.VMEM((128,), jnp.float32),
                pltpu.VMEM((l,), jnp.float32),
                pltpu.VMEM((l,), jnp.float32),
                pltpu.VMEM((128,), jnp.float32),
                pltpu.VMEM((l + 2 * _L16,), jnp.float32),
                pltpu.VMEM((l + 2 * _L16,), jnp.float32),
                pltpu.VMEM((_L16,), jnp.float32),
                pltpu.SemaphoreType.DMA,
                pltpu.SemaphoreType.DMA,
                pltpu.SemaphoreType.DMA,
                pltpu.SemaphoreType.DMA,
                pltpu.SemaphoreType.DMA,
                pltpu.SemaphoreType.DMA,
            ],
        )(functools.partial(_sc_body, chunk_base=c * cb, chunk_rows=cb))
        partials.append(fn(output, target, thr_c))

    return sum(jnp.sum(p) for p in partials) / (b * _NP * _NN)


# speedup vs baseline: 1.0614x; 1.0614x over previous
import functools

import jax
import jax.numpy as jnp
from jax import lax
from jax.experimental import pallas as pl
from jax.experimental.pallas import tpu as pltpu
from jax.experimental.pallas import tpu_sc as plsc

_MXL = 100.0
_MNL = -100.0
_NP = 8
_NN = 64
_BIG = 3.4e38
_L16 = 16


def _vec16(x):
    return jnp.full((_L16,), x, jnp.float32)


_IOTA = lambda: lax.broadcasted_iota(jnp.int32, (_L16,), 0)


def _extract(v, j, pad):
    return jnp.min(jnp.where(_IOTA() == j, v, pad))


def _sc_body(x_hbm, t_hbm, thr_hbm, out_hbm, xbuf, tbuf, thrbuf,
             xbuf2, tbuf2, thrbuf2, nbuf, pbuf, obuf,
             semx, semt, semh, semx2, semt2, semh2,
             *, chunk_base, chunk_rows):
    l = x_hbm.shape[1]
    nvec = l // _L16
    wid = lax.axis_index("s") * 2 + lax.axis_index("c")
    rpw = chunk_rows // 32
    iota = _IOTA()

    def row_compute(acc, xbuf, tbuf, thrbuf):
        thv = thrbuf[pl.ds(0, _L16)]
        tau_n = jnp.min(jnp.where(iota == 0, thv, _BIG))
        tau_p = jnp.min(jnp.where(iota == 1, thv, _BIG))

        nvec2 = nvec // 2
        stride = (l >> 1) + _L16

        @plsc.parallel_loop(0, nvec2, unroll=8,
                            carry=(jnp.int32(0),) * 4)
        def p2(i, carry):
            offs = list(carry)
            for h in range(2):
                xv = xbuf[pl.ds((h * nvec2 + i) * _L16, _L16)]
                tv = tbuf[pl.ds((h * nvec2 + i) * _L16, _L16)]
                m1 = tv == 1.0
                sm = jnp.where(m1, _MNL, xv)
                sp = jnp.where(m1, xv, _MXL)
                mn = sm >= tau_n
                mp = sp <= tau_p
                plsc.store_compressed(
                    nbuf.at[pl.ds(h * stride + offs[2 * h], _L16)],
                    sm, mask=mn)
                plsc.store_compressed(
                    pbuf.at[pl.ds(h * stride + offs[2 * h + 1], _L16)],
                    sp, mask=mp)
                offs[2 * h] += plsc.all_reduce_population_count(mn)[0]
                offs[2 * h + 1] += plsc.all_reduce_population_count(mp)[0]
            return tuple(offs)

        cn0, cp0, cn1, cp1 = p2
        nbuf[pl.ds(cn0, _L16)] = _vec16(-_BIG)
        pbuf[pl.ds(cp0, _L16)] = _vec16(_BIG)
        nbuf[pl.ds(stride + cn1, _L16)] = _vec16(-_BIG)
        pbuf[pl.ds(stride + cp1, _L16)] = _vec16(_BIG)

        def casc_n(base):
            def step(i, ts):
                s = jnp.sort(-nbuf[pl.ds(base + i * _L16, _L16)])
                for k in range(4):
                    r = lax.rev(s, (0,))
                    lo = jnp.minimum(ts[k], r)
                    hi = jnp.maximum(ts[k], r)
                    ts = ts[:k] + (jnp.sort(lo),) + ts[k + 1:]
                    s = jnp.sort(hi)
                return ts
            return step

        tn = lax.fori_loop(0, (cn0 + _L16 - 1) >> 4, casc_n(0),
                           (_vec16(_BIG),) * 4)
        tn = lax.fori_loop(0, (cn1 + _L16 - 1) >> 4, casc_n(stride), tn)

        def casc_p(base):
            def step(i, t0):
                s = jnp.sort(pbuf[pl.ds(base + i * _L16, _L16)])
                return jnp.sort(jnp.minimum(t0, lax.rev(s, (0,))))
            return step

        tp = lax.fori_loop(0, (cp0 + _L16 - 1) >> 4, casc_p(0), _vec16(_BIG))
        tp = lax.fori_loop(0, (cp1 + _L16 - 1) >> 4, casc_p(stride), tp)

        maxneg = -jnp.min(tn[0])
        for j in range(_NP):
            sp = _extract(tp, j, _BIG)
            mp_ = jnp.maximum(maxneg + 1.0 - sp, 0.0)
            num_v = _vec16(0.0)
            den_v = _vec16(0.0)
            for k in range(4):
                lossv = jnp.maximum(-tn[k] + (1.0 - sp), 0.0)
                prob = jnp.where(lossv > 0.0, lossv, _MNL)
                e = jnp.exp(prob - mp_)
                num_v = num_v + e * lossv
                den_v = den_v + e
            num = jnp.sum(num_v)
            den = jnp.sum(den_v)
            q = jnp.where(iota == 0, num, 0.0) / jnp.where(iota == 0, den, 1.0)
            acc = acc + jnp.where((iota == 0) & (num > 0.0), q, 0.0)
        return acc

    def start(row, xb, tb, hb, sx, st, sh):
        pltpu.make_async_copy(x_hbm.at[chunk_base + row], xb, sx).start()
        pltpu.make_async_copy(t_hbm.at[chunk_base + row], tb, st).start()
        pltpu.make_async_copy(thr_hbm.at[row], hb, sh).start()

    def wait(row, xb, tb, hb, sx, st, sh):
        pltpu.make_async_copy(x_hbm.at[chunk_base + row], xb, sx).wait()
        pltpu.make_async_copy(t_hbm.at[chunk_base + row], tb, st).wait()
        pltpu.make_async_copy(thr_hbm.at[row], hb, sh).wait()

    base = wid * rpw
    start(base, xbuf, tbuf, thrbuf, semx, semt, semh)

    def pair_step(k, acc):
        ra = base + 2 * k
        rb = ra + 1
        start(rb, xbuf2, tbuf2, thrbuf2, semx2, semt2, semh2)
        wait(ra, xbuf, tbuf, thrbuf, semx, semt, semh)
        acc = row_compute(acc, xbuf, tbuf, thrbuf)
        start(jnp.minimum(ra + 2, base + rpw - 1), xbuf, tbuf, thrbuf,
              semx, semt, semh)
        wait(rb, xbuf2, tbuf2, thrbuf2, semx2, semt2, semh2)
        return row_compute(acc, xbuf2, tbuf2, thrbuf2)

    acc = lax.fori_loop(0, rpw // 2, pair_step, _vec16(0.0))
    wait(base, xbuf, tbuf, thrbuf, semx, semt, semh)
    obuf[...] = acc
    pltpu.sync_copy(obuf, out_hbm.at[pl.ds(wid * _L16, _L16)])


def _thr_body(out_ref, tgt_ref, thr_ref):
    x = out_ref[...]
    r, l = x.shape
    tmask = tgt_ref[...] == 1.0
    sim_m = jnp.where(tmask, _MNL, x)
    sim_p = jnp.where(tmask, x, _MXL)
    tau_n = jnp.min(jnp.max(sim_m.reshape(r, 64, l // 64), axis=2), axis=1)
    tau_p = jnp.max(jnp.min(sim_p.reshape(r, 8, l // 8), axis=2), axis=1)
    lane = lax.broadcasted_iota(jnp.int32, (r, 128), 1)
    thr_ref[...] = jnp.where(lane == 0, tau_n[:, None],
                             jnp.where(lane == 1, tau_p[:, None], 0.0))


_NCHUNK = 4


@jax.jit
def kernel(output, target):
    b, l = output.shape
    r = 16
    cb = b // _NCHUNK
    mesh = plsc.VectorSubcoreMesh(core_axis_name="c", subcore_axis_name="s")

    partials = []
    for c in range(_NCHUNK):
        thr_c = pl.pallas_call(
            _thr_body,
            grid=(cb // r,),
            in_specs=[
                pl.BlockSpec((r, l), lambda i, _c=c: (i + _c * (cb // r), 0)),
                pl.BlockSpec((r, l), lambda i, _c=c: (i + _c * (cb // r), 0)),
            ],
            out_specs=pl.BlockSpec((r, 128), lambda i: (i, 0)),
            out_shape=jax.ShapeDtypeStruct((cb, 128), jnp.float32),
        )(output, target)

        fn = functools.partial(
            pl.kernel,
            mesh=mesh,
            out_type=jax.ShapeDtypeStruct((32 * _L16,), jnp.float32),
            compiler_params=pltpu.CompilerParams(needs_layout_passes=False),
            scratch_types=[
                pltpu.VMEM((l,), jnp.float32),
                pltpu.VMEM((l,), jnp.float32),
                pltpu.VMEM((128,), jnp.float32),
                pltpu.VMEM((l,), jnp.float32),
                pltpu.VMEM((l,), jnp.float32),
                pltpu.VMEM((128,), jnp.float32),
                pltpu.VMEM((l + 2 * _L16,), jnp.float32),
                pltpu.VMEM((l + 2 * _L16,), jnp.float32),
                pltpu.VMEM((_L16,), jnp.float32),
                pltpu.SemaphoreType.DMA,
                pltpu.SemaphoreType.DMA,
                pltpu.SemaphoreType.DMA,
                pltpu.SemaphoreType.DMA,
                pltpu.SemaphoreType.DMA,
                pltpu.SemaphoreType.DMA,
            ],
        )(functools.partial(_sc_body, chunk_base=c * cb, chunk_rows=cb))
        partials.append(fn(output, target, thr_c))

    return sum(jnp.sum(p) for p in partials) / (b * _NP * _NN)
